# Initial kernel scaffold; baseline (speedup 1.0000x reference)
import jax
import jax.numpy as jnp
from jax.experimental import pallas as pl


def kernel(x, pos, W1a, b1a, W1b, b1b, W2a, b2a, W2b, b2b, W3a, b3a, W3b, b3b):
    raise NotImplementedError("write your pallas kernel here")


# speedup vs baseline: 6.2793x; 6.2793x over previous
import functools

import jax
import jax.numpy as jnp
from jax.experimental import pallas as pl
from jax.experimental.pallas import tpu as pltpu

_MAXK = 128
_BISECT_ITERS = 46


def _fps_body(px_ref, py_ref, qx_ref, qy_ref, *, M):
    px = px_ref[...]
    py = py_ref[...]
    B, N = px.shape
    iota_n = jax.lax.broadcasted_iota(jnp.int32, (1, N), 1)
    iota_m = jax.lax.broadcasted_iota(jnp.int32, (1, M), 1)
    lastx = px[:, 0:1]
    lasty = py[:, 0:1]
    sel0 = iota_m == 0
    qx = jnp.where(sel0, lastx, 0.0)
    qy = jnp.where(sel0, lasty, 0.0)
    dist0 = jnp.full((B, N), jnp.inf, dtype=jnp.float32)

    def body(i, carry):
        dist, lx, ly, qx, qy = carry
        d = (px - lx) ** 2 + (py - ly) ** 2
        dist = jnp.minimum(dist, d)
        m = jnp.max(dist, axis=1, keepdims=True)
        idx = jnp.min(jnp.where(dist == m, iota_n, N), axis=1, keepdims=True)
        selp = iota_n == idx
        nx = jnp.sum(jnp.where(selp, px, 0.0), axis=1, keepdims=True)
        ny = jnp.sum(jnp.where(selp, py, 0.0), axis=1, keepdims=True)
        selq = iota_m == i
        qx = jnp.where(selq, nx, qx)
        qy = jnp.where(selq, ny, qy)
        return dist, nx, ny, qx, qy

    _, _, _, qx, qy = jax.lax.fori_loop(1, M, body, (dist0, lastx, lasty, qx, qy))
    qx_ref[...] = qx
    qy_ref[...] = qy


def _fps(px, py, M):
    B, N = px.shape
    return pl.pallas_call(
        functools.partial(_fps_body, M=M),
        out_shape=[
            jax.ShapeDtypeStruct((B, M), jnp.float32),
            jax.ShapeDtypeStruct((B, M), jnp.float32),
        ],
    )(px, py)


def _sa_body(x_ref, pos_ref, q_ref, qT_ref, wax_ref, wap_ref, ba_ref,
             wb_ref, bb_ref, out_ref, v_ref, *, r2):
    X = x_ref[0]
    P = pos_ref[0]
    Q = q_ref[0]
    QT = qT_ref[0]
    N = X.shape[0]
    M = QT.shape[1]
    f32 = jnp.float32

    PU = (jnp.dot(X, wax_ref[...], preferred_element_type=f32)
          + jnp.dot(P, wap_ref[...], preferred_element_type=f32)
          + ba_ref[...])
    v_ref[...] = jnp.dot(Q, wap_ref[...], preferred_element_type=f32)

    Px = P[:, 0:1]
    Py = P[:, 1:2]
    QTx = QT[0:1, :]
    QTy = QT[1:2, :]
    ddx = Px - QTx
    ddy = Py - QTy
    DT = ddx * ddx + ddy * ddy

    cnt = jnp.sum((DT <= r2).astype(jnp.int32), axis=0, keepdims=True)

    def bis(_, c):
        lo, hi = c
        mid = 0.5 * (lo + hi)
        cm = jnp.sum((DT <= mid).astype(jnp.int32), axis=0, keepdims=True)
        ge = cm >= _MAXK
        return jnp.where(ge, lo, mid), jnp.where(ge, mid, hi)

    lo0 = jnp.zeros((1, M), f32)
    hi0 = jnp.full((1, M), r2, f32)
    _, hi = jax.lax.fori_loop(0, _BISECT_ITERS, bis, (lo0, hi0))
    thresh = jnp.where(cnt > _MAXK, hi, jnp.full((1, M), r2, f32))

    Wb = wb_ref[...]
    bb = bb_ref[...]
    iota_m = jax.lax.broadcasted_iota(jnp.int32, (1, M), 1)

    def qloop(q, _):
        sel = iota_m == q
        qx = jnp.sum(jnp.where(sel, QTx, 0.0))
        qy = jnp.sum(jnp.where(sel, QTy, 0.0))
        th = jnp.sum(jnp.where(sel, thresh, 0.0))
        ex = Px - qx
        ey = Py - qy
        d2c = ex * ex + ey * ey
        bias = jnp.where(d2c <= th, 0.0, -1e30)
        vrow = v_ref[pl.ds(q, 1), :]
        t = jnp.tanh(PU - vrow)
        h = jnp.dot(t, Wb, preferred_element_type=f32) + bb
        r = jnp.max(h + bias, axis=0, keepdims=True)
        out_ref[0, pl.ds(q, 1), :] = r
        return 0

    jax.lax.fori_loop(0, M, qloop, 0)


def _sa(X, pos, q, qT, Wa, ba, Wb, bb, r2):
    B, N, F = X.shape
    M = qT.shape[2]
    Co2 = Wb.shape[1]
    wax = Wa[:F]
    wap = Wa[F:]
    ba2 = ba.reshape(1, -1)
    bb2 = bb.reshape(1, -1)
    return pl.pallas_call(
        functools.partial(_sa_body, r2=r2),
        grid=(B,),
        in_specs=[
            pl.BlockSpec((1, N, F), lambda b: (b, 0, 0)),
            pl.BlockSpec((1, N, 2), lambda b: (b, 0, 0)),
            pl.BlockSpec((1, M, 2), lambda b: (b, 0, 0)),
            pl.BlockSpec((1, 2, M), lambda b: (b, 0, 0)),
            pl.BlockSpec(wax.shape, lambda b: (0, 0)),
            pl.BlockSpec(wap.shape, lambda b: (0, 0)),
            pl.BlockSpec(ba2.shape, lambda b: (0, 0)),
            pl.BlockSpec(Wb.shape, lambda b: (0, 0)),
            pl.BlockSpec(bb2.shape, lambda b: (0, 0)),
        ],
        out_specs=pl.BlockSpec((1, M, Co2), lambda b: (b, 0, 0)),
        out_shape=jax.ShapeDtypeStruct((B, M, Co2), jnp.float32),
        scratch_shapes=[
            pltpu.VMEM((M, Wb.shape[0]), jnp.float32),
        ],
    )(X, pos, q, qT, wax, wap, ba2, Wb, bb2)


def _glob_body(x_ref, q_ref, wax_ref, wap_ref, ba_ref, wb_ref, bb_ref, out_ref):
    f32 = jnp.float32
    X = x_ref[0]
    Q = q_ref[0]
    h = jnp.tanh(jnp.dot(X, wax_ref[...], preferred_element_type=f32)
                 + jnp.dot(Q, wap_ref[...], preferred_element_type=f32)
                 + ba_ref[...])
    o = jnp.dot(h, wb_ref[...], preferred_element_type=f32) + bb_ref[...]
    out_ref[0] = jnp.max(o, axis=0, keepdims=True)


def _glob(X, Q, Wa, ba, Wb, bb):
    B, M, C = X.shape
    Co2 = Wb.shape[1]
    wax = Wa[:C]
    wap = Wa[C:]
    ba2 = ba.reshape(1, -1)
    bb2 = bb.reshape(1, -1)
    return pl.pallas_call(
        _glob_body,
        grid=(B,),
        in_specs=[
            pl.BlockSpec((1, M, C), lambda b: (b, 0, 0)),
            pl.BlockSpec((1, M, 2), lambda b: (b, 0, 0)),
            pl.BlockSpec(wax.shape, lambda b: (0, 0)),
            pl.BlockSpec(wap.shape, lambda b: (0, 0)),
            pl.BlockSpec(ba2.shape, lambda b: (0, 0)),
            pl.BlockSpec(Wb.shape, lambda b: (0, 0)),
            pl.BlockSpec(bb2.shape, lambda b: (0, 0)),
        ],
        out_specs=pl.BlockSpec((1, 1, Co2), lambda b: (b, 0, 0)),
        out_shape=jax.ShapeDtypeStruct((B, 1, Co2), jnp.float32),
    )(X, Q, wax, wap, ba2, Wb, bb2)[:, 0, :]


def kernel(x, pos, W1a, b1a, W1b, b1b, W2a, b2a, W2b, b2b, W3a, b3a, W3b, b3b):
    B, N, _ = x.shape
    M1 = N // 4
    M2 = M1 // 4
    r1sq = float(0.4 * 0.4)
    r2sq = float(0.8 * 0.8)

    px = pos[:, :, 0]
    py = pos[:, :, 1]

    q1x, q1y = _fps(px, py, M1)
    q1 = jnp.stack([q1x, q1y], axis=-1)
    q1T = jnp.stack([q1x, q1y], axis=1)

    x1 = _sa(x, pos, q1, q1T, W1a, b1a, W1b, b1b, r2=r1sq)

    q2x, q2y = _fps(q1x, q1y, M2)
    q2 = jnp.stack([q2x, q2y], axis=-1)
    q2T = jnp.stack([q2x, q2y], axis=1)

    x2 = _sa(x1, q1, q2, q2T, W2a, b2a, W2b, b2b, r2=r2sq)

    return _glob(x2, q2, W3a, b3a, W3b, b3b)


# speedup vs baseline: 6.3051x; 1.0041x over previous
import functools

import jax
import jax.numpy as jnp
from jax.experimental import pallas as pl
from jax.experimental.pallas import tpu as pltpu

_MAXK = 128
_BISECT_ITERS = 46


def _fps_body(px_ref, py_ref, qx_ref, qy_ref, *, M):
    px = px_ref[...]
    py = py_ref[...]
    B, N = px.shape
    iota_n = jax.lax.broadcasted_iota(jnp.int32, (1, N), 1)
    iota_m = jax.lax.broadcasted_iota(jnp.int32, (1, M), 1)
    lastx = px[:, 0:1]
    lasty = py[:, 0:1]
    sel0 = iota_m == 0
    qx = jnp.where(sel0, lastx, 0.0)
    qy = jnp.where(sel0, lasty, 0.0)
    dist0 = jnp.full((B, N), jnp.inf, dtype=jnp.float32)

    def body(i, carry):
        dist, lx, ly, qx, qy = carry
        d = (px - lx) ** 2 + (py - ly) ** 2
        dist = jnp.minimum(dist, d)
        m = jnp.max(dist, axis=1, keepdims=True)
        idx = jnp.min(jnp.where(dist == m, iota_n, N), axis=1, keepdims=True)
        selp = iota_n == idx
        nx = jnp.sum(jnp.where(selp, px, 0.0), axis=1, keepdims=True)
        ny = jnp.sum(jnp.where(selp, py, 0.0), axis=1, keepdims=True)
        selq = iota_m == i
        qx = jnp.where(selq, nx, qx)
        qy = jnp.where(selq, ny, qy)
        return dist, nx, ny, qx, qy

    _, _, _, qx, qy = jax.lax.fori_loop(1, M, body, (dist0, lastx, lasty, qx, qy))
    qx_ref[...] = qx
    qy_ref[...] = qy


def _fps(px, py, M):
    B, N = px.shape
    return pl.pallas_call(
        functools.partial(_fps_body, M=M),
        out_shape=[
            jax.ShapeDtypeStruct((B, M), jnp.float32),
            jax.ShapeDtypeStruct((B, M), jnp.float32),
        ],
    )(px, py)


def _sa_body(x_ref, pos_ref, q_ref, qT_ref, wax_ref, wap_ref, ba_ref,
             wb_ref, bb_ref, out_ref, v_ref, *, r2):
    X = x_ref[0]
    P = pos_ref[0]
    Q = q_ref[0]
    QT = qT_ref[0]
    N = X.shape[0]
    M = QT.shape[1]
    f32 = jnp.float32

    PU = (jnp.dot(X, wax_ref[...], preferred_element_type=f32)
          + jnp.dot(P, wap_ref[...], preferred_element_type=f32)
          + ba_ref[...])
    v_ref[...] = jnp.dot(Q, wap_ref[...], preferred_element_type=f32)

    Px = P[:, 0:1]
    Py = P[:, 1:2]
    QTx = QT[0:1, :]
    QTy = QT[1:2, :]
    ddx = Px - QTx
    ddy = Py - QTy
    DT = ddx * ddx + ddy * ddy

    cnt = jnp.sum((DT <= r2).astype(jnp.int32), axis=0, keepdims=True)

    def bis(_, c):
        lo, hi = c
        mid = 0.5 * (lo + hi)
        cm = jnp.sum((DT <= mid).astype(jnp.int32), axis=0, keepdims=True)
        ge = cm >= _MAXK
        return jnp.where(ge, lo, mid), jnp.where(ge, mid, hi)

    lo0 = jnp.zeros((1, M), f32)
    hi0 = jnp.full((1, M), r2, f32)
    _, hi = jax.lax.fori_loop(0, _BISECT_ITERS, bis, (lo0, hi0))
    thresh = jnp.where(cnt > _MAXK, hi, jnp.full((1, M), r2, f32))

    Wb = wb_ref[...].astype(jnp.bfloat16)
    bb = bb_ref[...]
    iota_m = jax.lax.broadcasted_iota(jnp.int32, (1, M), 1)

    def qloop(q, _):
        sel = iota_m == q
        qx = jnp.sum(jnp.where(sel, QTx, 0.0))
        qy = jnp.sum(jnp.where(sel, QTy, 0.0))
        th = jnp.sum(jnp.where(sel, thresh, 0.0))
        ex = Px - qx
        ey = Py - qy
        d2c = ex * ex + ey * ey
        bias = jnp.where(d2c <= th, 0.0, -1e30)
        vrow = v_ref[pl.ds(q, 1), :]
        t = jnp.tanh(PU - vrow).astype(jnp.bfloat16)
        h = jnp.dot(t, Wb, preferred_element_type=f32)
        r = jnp.max(h + bias, axis=0, keepdims=True) + bb
        out_ref[0, pl.ds(q, 1), :] = r
        return 0

    jax.lax.fori_loop(0, M, qloop, 0)


def _sa(X, pos, q, qT, Wa, ba, Wb, bb, r2):
    B, N, F = X.shape
    M = qT.shape[2]
    Co2 = Wb.shape[1]
    wax = Wa[:F]
    wap = Wa[F:]
    ba2 = ba.reshape(1, -1)
    bb2 = bb.reshape(1, -1)
    return pl.pallas_call(
        functools.partial(_sa_body, r2=r2),
        grid=(B,),
        in_specs=[
            pl.BlockSpec((1, N, F), lambda b: (b, 0, 0)),
            pl.BlockSpec((1, N, 2), lambda b: (b, 0, 0)),
            pl.BlockSpec((1, M, 2), lambda b: (b, 0, 0)),
            pl.BlockSpec((1, 2, M), lambda b: (b, 0, 0)),
            pl.BlockSpec(wax.shape, lambda b: (0, 0)),
            pl.BlockSpec(wap.shape, lambda b: (0, 0)),
            pl.BlockSpec(ba2.shape, lambda b: (0, 0)),
            pl.BlockSpec(Wb.shape, lambda b: (0, 0)),
            pl.BlockSpec(bb2.shape, lambda b: (0, 0)),
        ],
        out_specs=pl.BlockSpec((1, M, Co2), lambda b: (b, 0, 0)),
        out_shape=jax.ShapeDtypeStruct((B, M, Co2), jnp.float32),
        scratch_shapes=[
            pltpu.VMEM((M, Wb.shape[0]), jnp.float32),
        ],
    )(X, pos, q, qT, wax, wap, ba2, Wb, bb2)


def _glob_body(x_ref, q_ref, wax_ref, wap_ref, ba_ref, wb_ref, bb_ref, out_ref):
    f32 = jnp.float32
    X = x_ref[0]
    Q = q_ref[0]
    h = jnp.tanh(jnp.dot(X, wax_ref[...], preferred_element_type=f32)
                 + jnp.dot(Q, wap_ref[...], preferred_element_type=f32)
                 + ba_ref[...])
    o = jnp.dot(h, wb_ref[...], preferred_element_type=f32) + bb_ref[...]
    out_ref[0] = jnp.max(o, axis=0, keepdims=True)


def _glob(X, Q, Wa, ba, Wb, bb):
    B, M, C = X.shape
    Co2 = Wb.shape[1]
    wax = Wa[:C]
    wap = Wa[C:]
    ba2 = ba.reshape(1, -1)
    bb2 = bb.reshape(1, -1)
    return pl.pallas_call(
        _glob_body,
        grid=(B,),
        in_specs=[
            pl.BlockSpec((1, M, C), lambda b: (b, 0, 0)),
            pl.BlockSpec((1, M, 2), lambda b: (b, 0, 0)),
            pl.BlockSpec(wax.shape, lambda b: (0, 0)),
            pl.BlockSpec(wap.shape, lambda b: (0, 0)),
            pl.BlockSpec(ba2.shape, lambda b: (0, 0)),
            pl.BlockSpec(Wb.shape, lambda b: (0, 0)),
            pl.BlockSpec(bb2.shape, lambda b: (0, 0)),
        ],
        out_specs=pl.BlockSpec((1, 1, Co2), lambda b: (b, 0, 0)),
        out_shape=jax.ShapeDtypeStruct((B, 1, Co2), jnp.float32),
    )(X, Q, wax, wap, ba2, Wb, bb2)[:, 0, :]


def kernel(x, pos, W1a, b1a, W1b, b1b, W2a, b2a, W2b, b2b, W3a, b3a, W3b, b3b):
    B, N, _ = x.shape
    M1 = N // 4
    M2 = M1 // 4
    r1sq = float(0.4 * 0.4)
    r2sq = float(0.8 * 0.8)

    px = pos[:, :, 0]
    py = pos[:, :, 1]

    q1x, q1y = _fps(px, py, M1)
    q1 = jnp.stack([q1x, q1y], axis=-1)
    q1T = jnp.stack([q1x, q1y], axis=1)

    x1 = _sa(x, pos, q1, q1T, W1a, b1a, W1b, b1b, r2=r1sq)

    q2x, q2y = _fps(q1x, q1y, M2)
    q2 = jnp.stack([q2x, q2y], axis=-1)
    q2T = jnp.stack([q2x, q2y], axis=1)

    x2 = _sa(x1, q1, q2, q2T, W2a, b2a, W2b, b2b, r2=r2sq)

    return _glob(x2, q2, W3a, b3a, W3b, b3b)


# speedup vs baseline: 6.3487x; 1.0069x over previous
import functools

import jax
import jax.numpy as jnp
from jax.experimental import pallas as pl
from jax.experimental.pallas import tpu as pltpu

_MAXK = 128
_BISECT_ITERS = 46


def _fps_body(px_ref, py_ref, qx_ref, qy_ref, *, M):
    px = px_ref[...]
    py = py_ref[...]
    B, N = px.shape
    iota_n = jax.lax.broadcasted_iota(jnp.int32, (1, N), 1)
    iota_m = jax.lax.broadcasted_iota(jnp.int32, (1, M), 1)
    lastx = px[:, 0:1]
    lasty = py[:, 0:1]
    sel0 = iota_m == 0
    qx = jnp.where(sel0, lastx, 0.0)
    qy = jnp.where(sel0, lasty, 0.0)
    dist0 = jnp.full((B, N), jnp.inf, dtype=jnp.float32)

    def body(i, carry):
        dist, lx, ly, qx, qy = carry
        d = (px - lx) ** 2 + (py - ly) ** 2
        dist = jnp.minimum(dist, d)
        m = jnp.max(dist, axis=1, keepdims=True)
        idx = jnp.min(jnp.where(dist == m, iota_n, N), axis=1, keepdims=True)
        selp = iota_n == idx
        nx = jnp.sum(jnp.where(selp, px, 0.0), axis=1, keepdims=True)
        ny = jnp.sum(jnp.where(selp, py, 0.0), axis=1, keepdims=True)
        selq = iota_m == i
        qx = jnp.where(selq, nx, qx)
        qy = jnp.where(selq, ny, qy)
        return dist, nx, ny, qx, qy

    _, _, _, qx, qy = jax.lax.fori_loop(1, M, body, (dist0, lastx, lasty, qx, qy))
    qx_ref[...] = qx
    qy_ref[...] = qy


def _fps(px, py, M):
    B, N = px.shape
    return pl.pallas_call(
        functools.partial(_fps_body, M=M),
        out_shape=[
            jax.ShapeDtypeStruct((B, M), jnp.float32),
            jax.ShapeDtypeStruct((B, M), jnp.float32),
        ],
    )(px, py)


def _sa_body(x_ref, pos_ref, q_ref, qT_ref, wax_ref, wap_ref, ba_ref,
             wb_ref, bb_ref, out_ref, v_ref, *, r2):
    X = x_ref[0]
    P = pos_ref[0]
    Q = q_ref[0]
    QT = qT_ref[0]
    N = X.shape[0]
    M = QT.shape[1]
    f32 = jnp.float32

    PU = (jnp.dot(X, wax_ref[...], preferred_element_type=f32)
          + jnp.dot(P, wap_ref[...], preferred_element_type=f32)
          + ba_ref[...])
    v_ref[...] = jnp.dot(Q, wap_ref[...], preferred_element_type=f32)

    Px = P[:, 0:1]
    Py = P[:, 1:2]
    QTx = QT[0:1, :]
    QTy = QT[1:2, :]
    ddx = Px - QTx
    ddy = Py - QTy
    DT = ddx * ddx + ddy * ddy

    cnt = jnp.sum((DT <= r2).astype(jnp.int32), axis=0, keepdims=True)

    def bis(_, c):
        lo, hi = c
        mid = 0.5 * (lo + hi)
        cm = jnp.sum((DT <= mid).astype(jnp.int32), axis=0, keepdims=True)
        ge = cm >= _MAXK
        return jnp.where(ge, lo, mid), jnp.where(ge, mid, hi)

    lo0 = jnp.zeros((1, M), f32)
    hi0 = jnp.full((1, M), r2, f32)
    _, hi = jax.lax.fori_loop(0, _BISECT_ITERS, bis, (lo0, hi0))
    thresh = jnp.where(cnt > _MAXK, hi, jnp.full((1, M), r2, f32))

    Wb = wb_ref[...].astype(jnp.bfloat16)
    bb = bb_ref[...]
    iota_m = jax.lax.broadcasted_iota(jnp.int32, (1, M), 1)

    def qloop(q, _):
        sel = iota_m == q
        qx = jnp.sum(jnp.where(sel, QTx, 0.0))
        qy = jnp.sum(jnp.where(sel, QTy, 0.0))
        th = jnp.sum(jnp.where(sel, thresh, 0.0))
        ex = Px - qx
        ey = Py - qy
        d2c = ex * ex + ey * ey
        bias = jnp.where(d2c <= th, 0.0, -1e30)
        vrow = v_ref[pl.ds(q, 1), :]
        t = jnp.tanh(PU - vrow).astype(jnp.bfloat16)
        h = jnp.dot(t, Wb, preferred_element_type=f32)
        r = jnp.max(h + bias, axis=0, keepdims=True) + bb
        out_ref[0, pl.ds(q, 1), :] = r
        return 0

    jax.lax.fori_loop(0, M, qloop, 0)


def _sa(X, pos, q, qT, Wa, ba, Wb, bb, r2):
    B, N, F = X.shape
    M = qT.shape[2]
    Co2 = Wb.shape[1]
    wax = Wa[:F]
    wap = Wa[F:]
    ba2 = ba.reshape(1, -1)
    bb2 = bb.reshape(1, -1)
    return pl.pallas_call(
        functools.partial(_sa_body, r2=r2),
        grid=(B,),
        in_specs=[
            pl.BlockSpec((1, N, F), lambda b: (b, 0, 0)),
            pl.BlockSpec((1, N, 2), lambda b: (b, 0, 0)),
            pl.BlockSpec((1, M, 2), lambda b: (b, 0, 0)),
            pl.BlockSpec((1, 2, M), lambda b: (b, 0, 0)),
            pl.BlockSpec(wax.shape, lambda b: (0, 0)),
            pl.BlockSpec(wap.shape, lambda b: (0, 0)),
            pl.BlockSpec(ba2.shape, lambda b: (0, 0)),
            pl.BlockSpec(Wb.shape, lambda b: (0, 0)),
            pl.BlockSpec(bb2.shape, lambda b: (0, 0)),
        ],
        out_specs=pl.BlockSpec((1, M, Co2), lambda b: (b, 0, 0)),
        out_shape=jax.ShapeDtypeStruct((B, M, Co2), jnp.float32),
        scratch_shapes=[
            pltpu.VMEM((M, Wb.shape[0]), jnp.float32),
        ],
        compiler_params=pltpu.CompilerParams(
            dimension_semantics=("parallel",)),
    )(X, pos, q, qT, wax, wap, ba2, Wb, bb2)


def _glob_body(x_ref, q_ref, wax_ref, wap_ref, ba_ref, wb_ref, bb_ref, out_ref):
    f32 = jnp.float32
    X = x_ref[0]
    Q = q_ref[0]
    h = jnp.tanh(jnp.dot(X, wax_ref[...], preferred_element_type=f32)
                 + jnp.dot(Q, wap_ref[...], preferred_element_type=f32)
                 + ba_ref[...])
    o = jnp.dot(h, wb_ref[...], preferred_element_type=f32) + bb_ref[...]
    out_ref[0] = jnp.max(o, axis=0, keepdims=True)


def _glob(X, Q, Wa, ba, Wb, bb):
    B, M, C = X.shape
    Co2 = Wb.shape[1]
    wax = Wa[:C]
    wap = Wa[C:]
    ba2 = ba.reshape(1, -1)
    bb2 = bb.reshape(1, -1)
    return pl.pallas_call(
        _glob_body,
        grid=(B,),
        in_specs=[
            pl.BlockSpec((1, M, C), lambda b: (b, 0, 0)),
            pl.BlockSpec((1, M, 2), lambda b: (b, 0, 0)),
            pl.BlockSpec(wax.shape, lambda b: (0, 0)),
            pl.BlockSpec(wap.shape, lambda b: (0, 0)),
            pl.BlockSpec(ba2.shape, lambda b: (0, 0)),
            pl.BlockSpec(Wb.shape, lambda b: (0, 0)),
            pl.BlockSpec(bb2.shape, lambda b: (0, 0)),
        ],
        out_specs=pl.BlockSpec((1, 1, Co2), lambda b: (b, 0, 0)),
        out_shape=jax.ShapeDtypeStruct((B, 1, Co2), jnp.float32),
    )(X, Q, wax, wap, ba2, Wb, bb2)[:, 0, :]


def kernel(x, pos, W1a, b1a, W1b, b1b, W2a, b2a, W2b, b2b, W3a, b3a, W3b, b3b):
    B, N, _ = x.shape
    M1 = N // 4
    M2 = M1 // 4
    r1sq = float(0.4 * 0.4)
    r2sq = float(0.8 * 0.8)

    px = pos[:, :, 0]
    py = pos[:, :, 1]

    q1x, q1y = _fps(px, py, M1)
    q1 = jnp.stack([q1x, q1y], axis=-1)
    q1T = jnp.stack([q1x, q1y], axis=1)

    x1 = _sa(x, pos, q1, q1T, W1a, b1a, W1b, b1b, r2=r1sq)

    q2x, q2y = _fps(q1x, q1y, M2)
    q2 = jnp.stack([q2x, q2y], axis=-1)
    q2T = jnp.stack([q2x, q2y], axis=1)

    x2 = _sa(x1, q1, q2, q2T, W2a, b2a, W2b, b2b, r2=r2sq)

    return _glob(x2, q2, W3a, b3a, W3b, b3b)
